# baseline (device time: 124779 ns/iter reference)
import jax
import jax.numpy as jnp
from jax import lax
from jax.experimental import pallas as pl
from jax.experimental.pallas import tpu as pltpu

T = 2048
TH = T // 2
D = 4096
V_LOCAL = 8192
VB = 1024
VBH = VB // 2
N_BLOCKS = V_LOCAL // VB


def _body(x_ref, w_ref, labels_ref, out_ref,
          la_ref, lb_ref, acc_ref, stats_ref, comm_ref,
          send_sem_y, recv_sem_y, send_sem_x, recv_sem_x):
    j = pl.program_id(0)
    my_x = lax.axis_index("x")
    my_y = lax.axis_index("y")
    x_bf_ref = x_ref

    @pl.when(j == 0)
    def _():
        acc_ref[...] = jnp.zeros_like(acc_ref)
        lb_ref[...] = jnp.zeros_like(lb_ref)

    col = lax.broadcasted_iota(jnp.int32, (VBH, 128), 1)
    e0 = (col == 0).astype(jnp.bfloat16)
    e1 = (col == 1).astype(jnp.bfloat16)

    def stats_update(b, logits):
        logits16 = logits.astype(jnp.bfloat16)
        p = jnp.exp(logits16)
        ids = (my_y * V_LOCAL + b * VBH) + lax.broadcasted_iota(
            jnp.int32, (TH, VBH), 1)
        sel = jnp.where(ids == labels_ref[...], logits16, jnp.bfloat16(0))
        acc_ref[...] += (
            jnp.dot(p, e0, preferred_element_type=jnp.float32)
            + jnp.dot(sel, e1, preferred_element_type=jnp.float32)
        )

    la_ref[...] = jnp.dot(
        x_bf_ref[...], w_ref[:, :VBH].astype(jnp.bfloat16),
        preferred_element_type=jnp.float32,
    )
    stats_update(2 * j - 1, lb_ref[...])
    lb_ref[...] = jnp.dot(
        x_bf_ref[...], w_ref[:, VBH:].astype(jnp.bfloat16),
        preferred_element_type=jnp.float32,
    )
    stats_update(2 * j, la_ref[...])

    @pl.when(j == N_BLOCKS - 1)
    def _():
        stats_update(2 * j + 1, lb_ref[...])
        stats_ref[...] = jnp.zeros_like(stats_ref)
        stats_ref[:, 0:1] = acc_ref[:, 0:1] - float(VBH)
        stats_ref[:, 1:2] = acc_ref[:, 1:2]

        nbr_y = (my_x, 1 - my_y)
        nbr_x = (1 - my_x, my_y)

        barrier_sem = pltpu.get_barrier_semaphore()
        for nbr in (nbr_y, nbr_x):
            pl.semaphore_signal(
                barrier_sem, inc=1, device_id=nbr,
                device_id_type=pl.DeviceIdType.MESH,
            )
        pl.semaphore_wait(barrier_sem, 2)

        rdma_y = pltpu.make_async_remote_copy(
            src_ref=stats_ref,
            dst_ref=comm_ref,
            send_sem=send_sem_y,
            recv_sem=recv_sem_y,
            device_id=nbr_y,
            device_id_type=pl.DeviceIdType.MESH,
        )
        rdma_y.start()
        rdma_y.wait()

        s_tot = stats_ref[:, 0:1] + comm_ref[:, 0:1]
        lab_tot = stats_ref[:, 1:2] + comm_ref[:, 1:2]
        out_ref[pl.ds(my_x * TH, TH), :] = jnp.log(s_tot) - lab_tot

        rdma_x = pltpu.make_async_remote_copy(
            src_ref=out_ref.at[pl.ds(my_x * TH, TH), :],
            dst_ref=out_ref.at[pl.ds(my_x * TH, TH), :],
            send_sem=send_sem_x,
            recv_sem=recv_sem_x,
            device_id=nbr_x,
            device_id_type=pl.DeviceIdType.MESH,
        )
        rdma_x.start()
        rdma_x.wait()


def kernel(x, W, labels):
    my_x = lax.axis_index("x")
    x_half = lax.dynamic_slice(x, (my_x * TH, 0), (TH, D)).astype(jnp.bfloat16)
    labels_half = lax.dynamic_slice(labels, (my_x * TH,), (TH,)).reshape(TH, 1)

    nll = pl.pallas_call(
        _body,
        grid=(N_BLOCKS,),
        in_specs=[
            pl.BlockSpec((TH, D), lambda j: (0, 0)),
            pl.BlockSpec((D, VB), lambda j: (0, j)),
            pl.BlockSpec((TH, 1), lambda j: (0, 0)),
        ],
        out_specs=pl.BlockSpec((T, 1), lambda j: (0, 0)),
        out_shape=jax.ShapeDtypeStruct((T, 1), jnp.float32),
        scratch_shapes=[
            pltpu.VMEM((TH, VBH), jnp.float32),
            pltpu.VMEM((TH, VBH), jnp.float32),
            pltpu.VMEM((TH, 128), jnp.float32),
            pltpu.VMEM((TH, 8), jnp.float32),
            pltpu.VMEM((TH, 8), jnp.float32),
            pltpu.SemaphoreType.DMA,
            pltpu.SemaphoreType.DMA,
            pltpu.SemaphoreType.DMA,
            pltpu.SemaphoreType.DMA,
        ],
        compiler_params=pltpu.CompilerParams(
            collective_id=0,
            vmem_limit_bytes=60 * 1024 * 1024,
        ),
    )(x_half, W, labels_half)

    return nll.reshape(T)


# device time: 112282 ns/iter; 1.1113x vs baseline; 1.1113x over previous
import jax
import jax.numpy as jnp
from jax import lax
from jax.experimental import pallas as pl
from jax.experimental.pallas import tpu as pltpu

T = 2048
TH = T // 2
D = 4096
V_LOCAL = 8192
VB = 512
VBH = VB // 2
N_BLOCKS = V_LOCAL // VB


def _body(mx_ref, x_ref, w_ref, labels_ref, out_ref,
          x_bf_ref, la_ref, lb_ref, acc_ref, stats_ref, comm_ref,
          send_sem_y, recv_sem_y, send_sem_x, recv_sem_x):
    j = pl.program_id(0)
    my_x = lax.axis_index("x")
    my_y = lax.axis_index("y")

    @pl.when(j == 0)
    def _():
        x_bf_ref[...] = x_ref[...].astype(jnp.bfloat16)
        acc_ref[...] = jnp.zeros_like(acc_ref)
        lb_ref[...] = jnp.zeros_like(lb_ref)

    def stats_update(b, logits):
        p = jnp.exp(logits)
        ids = (my_y * V_LOCAL + b * VBH) + lax.broadcasted_iota(
            jnp.int32, (TH, VBH), 1)
        sel = jnp.where(ids == labels_ref[...], logits, 0.0)
        acc_ref[:, 0:1] += jnp.sum(p, axis=1, keepdims=True)
        acc_ref[:, 1:2] += jnp.sum(sel, axis=1, keepdims=True)

    la_ref[...] = jnp.dot(
        x_bf_ref[...], w_ref[:, :VBH].astype(jnp.bfloat16),
        preferred_element_type=jnp.float32,
    )
    stats_update(2 * j - 1, lb_ref[...])
    lb_ref[...] = jnp.dot(
        x_bf_ref[...], w_ref[:, VBH:].astype(jnp.bfloat16),
        preferred_element_type=jnp.float32,
    )
    stats_update(2 * j, la_ref[...])

    @pl.when(j == N_BLOCKS - 1)
    def _():
        stats_update(2 * j + 1, lb_ref[...])
        stats_ref[...] = jnp.zeros_like(stats_ref)
        stats_ref[:, 0:1] = acc_ref[:, 0:1] - float(VBH)
        stats_ref[:, 1:2] = acc_ref[:, 1:2]

        nbr_y = (my_x, 1 - my_y)
        nbr_x = (1 - my_x, my_y)

        barrier_sem = pltpu.get_barrier_semaphore()
        for nbr in (nbr_y, nbr_x):
            pl.semaphore_signal(
                barrier_sem, inc=1, device_id=nbr,
                device_id_type=pl.DeviceIdType.MESH,
            )
        pl.semaphore_wait(barrier_sem, 2)

        rdma_y = pltpu.make_async_remote_copy(
            src_ref=stats_ref,
            dst_ref=comm_ref,
            send_sem=send_sem_y,
            recv_sem=recv_sem_y,
            device_id=nbr_y,
            device_id_type=pl.DeviceIdType.MESH,
        )
        rdma_y.start()
        rdma_y.wait()

        s_tot = stats_ref[:, 0:1] + comm_ref[:, 0:1]
        lab_tot = stats_ref[:, 1:2] + comm_ref[:, 1:2]
        out_ref[pl.ds(my_x * TH, TH), :] = jnp.log(s_tot) - lab_tot

        rdma_x = pltpu.make_async_remote_copy(
            src_ref=out_ref.at[pl.ds(my_x * TH, TH), :],
            dst_ref=out_ref.at[pl.ds(my_x * TH, TH), :],
            send_sem=send_sem_x,
            recv_sem=recv_sem_x,
            device_id=nbr_x,
            device_id_type=pl.DeviceIdType.MESH,
        )
        rdma_x.start()
        rdma_x.wait()


def kernel(x, W, labels):
    my_x = jnp.reshape(lax.axis_index("x"), (1,)).astype(jnp.int32)
    labels2d = labels.reshape(T, 1)

    grid_spec = pltpu.PrefetchScalarGridSpec(
        num_scalar_prefetch=1,
        grid=(N_BLOCKS,),
        in_specs=[
            pl.BlockSpec((TH, D), lambda j, mx: (mx[0], 0)),
            pl.BlockSpec((D, VB), lambda j, mx: (0, j)),
            pl.BlockSpec((TH, 1), lambda j, mx: (mx[0], 0)),
        ],
        out_specs=pl.BlockSpec((T, 1), lambda j, mx: (0, 0)),
        scratch_shapes=[
            pltpu.VMEM((TH, D), jnp.bfloat16),
            pltpu.VMEM((TH, VBH), jnp.float32),
            pltpu.VMEM((TH, VBH), jnp.float32),
            pltpu.VMEM((TH, 128), jnp.float32),
            pltpu.VMEM((TH, 8), jnp.float32),
            pltpu.VMEM((TH, 8), jnp.float32),
            pltpu.SemaphoreType.DMA,
            pltpu.SemaphoreType.DMA,
            pltpu.SemaphoreType.DMA,
            pltpu.SemaphoreType.DMA,
        ],
    )
    nll = pl.pallas_call(
        _body,
        grid_spec=grid_spec,
        out_shape=jax.ShapeDtypeStruct((T, 1), jnp.float32),
        compiler_params=pltpu.CompilerParams(
            collective_id=0,
            vmem_limit_bytes=60 * 1024 * 1024,
        ),
    )(my_x, x, W, labels2d)

    return nll.reshape(T)


# device time: 105251 ns/iter; 1.1855x vs baseline; 1.0668x over previous
import jax
import jax.numpy as jnp
from jax import lax
from jax.experimental import pallas as pl
from jax.experimental.pallas import tpu as pltpu

T = 2048
TH = T // 2
D = 4096
V_LOCAL = 8192
VB = 512
VBH = VB // 2
N_BLOCKS = V_LOCAL // VB


def _body(mx_ref, x_ref, w_ref, labels_ref, out_ref,
          x_bf_ref, la_ref, lb_ref, acc_ref, stats_ref, comm_ref,
          send_sem_y, recv_sem_y, send_sem_x, recv_sem_x):
    j = pl.program_id(0)
    my_x = lax.axis_index("x")
    my_y = lax.axis_index("y")

    @pl.when(j == 0)
    def _():
        x_bf_ref[...] = x_ref[...].astype(jnp.bfloat16)
        acc_ref[...] = jnp.zeros_like(acc_ref)
        lb_ref[...] = jnp.zeros_like(lb_ref)

    def stats_update(b, logits):
        p = jnp.exp(logits)
        ids = ((my_y * V_LOCAL + b * VBH)
               + lax.broadcasted_iota(jnp.int32, (TH, VBH), 1)
               ).astype(jnp.float32)
        sel = jnp.where(ids == labels_ref[...], logits, 0.0)
        acc_ref[:, 0:1] += jnp.sum(p, axis=1, keepdims=True)
        acc_ref[:, 1:2] += jnp.sum(sel, axis=1, keepdims=True)

    la_ref[...] = jnp.dot(
        x_bf_ref[...], w_ref[:, :VBH].astype(jnp.bfloat16),
        preferred_element_type=jnp.float32,
    )
    stats_update(2 * j - 1, lb_ref[...])
    lb_ref[...] = jnp.dot(
        x_bf_ref[...], w_ref[:, VBH:].astype(jnp.bfloat16),
        preferred_element_type=jnp.float32,
    )
    stats_update(2 * j, la_ref[...])

    @pl.when(j == N_BLOCKS - 1)
    def _():
        stats_update(2 * j + 1, lb_ref[...])
        stats_ref[...] = jnp.zeros_like(stats_ref)
        stats_ref[:, 0:1] = acc_ref[:, 0:1] - float(VBH)
        stats_ref[:, 1:2] = acc_ref[:, 1:2]

        nbr_y = (my_x, 1 - my_y)
        nbr_x = (1 - my_x, my_y)

        barrier_sem = pltpu.get_barrier_semaphore()
        for nbr in (nbr_y, nbr_x):
            pl.semaphore_signal(
                barrier_sem, inc=1, device_id=nbr,
                device_id_type=pl.DeviceIdType.MESH,
            )
        pl.semaphore_wait(barrier_sem, 2)

        rdma_y = pltpu.make_async_remote_copy(
            src_ref=stats_ref,
            dst_ref=comm_ref,
            send_sem=send_sem_y,
            recv_sem=recv_sem_y,
            device_id=nbr_y,
            device_id_type=pl.DeviceIdType.MESH,
        )
        rdma_y.start()
        rdma_y.wait()

        s_tot = stats_ref[:, 0:1] + comm_ref[:, 0:1]
        lab_tot = stats_ref[:, 1:2] + comm_ref[:, 1:2]
        nll = jnp.log(s_tot) - lab_tot
        out_ref[pl.ds(my_x * TH, TH)] = nll.reshape(TH)

        rdma_x = pltpu.make_async_remote_copy(
            src_ref=out_ref.at[pl.ds(my_x * TH, TH)],
            dst_ref=out_ref.at[pl.ds(my_x * TH, TH)],
            send_sem=send_sem_x,
            recv_sem=recv_sem_x,
            device_id=nbr_x,
            device_id_type=pl.DeviceIdType.MESH,
        )
        rdma_x.start()
        rdma_x.wait()


def kernel(x, W, labels):
    my_x = jnp.reshape(lax.axis_index("x"), (1,)).astype(jnp.int32)
    labels2d = labels.astype(jnp.float32).reshape(T, 1)

    grid_spec = pltpu.PrefetchScalarGridSpec(
        num_scalar_prefetch=1,
        grid=(N_BLOCKS,),
        in_specs=[
            pl.BlockSpec((TH, D), lambda j, mx: (mx[0], 0)),
            pl.BlockSpec((D, VB), lambda j, mx: (0, j)),
            pl.BlockSpec((TH, 1), lambda j, mx: (mx[0], 0)),
        ],
        out_specs=pl.BlockSpec((T,), lambda j, mx: (0,)),
        scratch_shapes=[
            pltpu.VMEM((TH, D), jnp.bfloat16),
            pltpu.VMEM((TH, VBH), jnp.float32),
            pltpu.VMEM((TH, VBH), jnp.float32),
            pltpu.VMEM((TH, 128), jnp.float32),
            pltpu.VMEM((TH, 8), jnp.float32),
            pltpu.VMEM((TH, 8), jnp.float32),
            pltpu.SemaphoreType.DMA,
            pltpu.SemaphoreType.DMA,
            pltpu.SemaphoreType.DMA,
            pltpu.SemaphoreType.DMA,
        ],
    )
    return pl.pallas_call(
        _body,
        grid_spec=grid_spec,
        out_shape=jax.ShapeDtypeStruct((T,), jnp.float32),
        compiler_params=pltpu.CompilerParams(
            collective_id=0,
            vmem_limit_bytes=60 * 1024 * 1024,
        ),
    )(my_x, x, W, labels2d)
